# Initial kernel scaffold; baseline (speedup 1.0000x reference)
import functools

import jax
import jax.numpy as jnp
from jax import lax
from jax.experimental import pallas as pl
from jax.experimental.pallas import tpu as pltpu
from jax.experimental.pallas import tpu_sc as plsc

N = 100000
K = 32
NC, NS, L = 2, 16, 16
NW = NC * NS
GROUPS = -(-N // (NW * L))
PER_TILE = GROUPS * L
NPAD = NW * PER_TILE
GPC = 7
CB = GPC * L
NCHUNK = GROUPS // GPC
EPC = CB * K * 2
STREAMS = EPC // 128


def _body(conn_hbm, table_hbm, outx_hbm, outy_hbm, outz_hbm,
          idx_v, rows_v, outx_v, outy_v, outz_v, sem_rows):
    wid = lax.axis_index("s") * NC + lax.axis_index("c")
    tile_base = wid * PER_TILE * K * 2

    jbase = lax.iota(jnp.int32, 16) * (K * 2)
    c0 = jnp.zeros((16,), jnp.int32)
    c1 = jnp.full((16,), 1, jnp.int32)
    c2 = jnp.full((16,), 2, jnp.int32)
    c3 = jnp.full((16,), 3, jnp.int32)
    ones = jnp.full((16,), 1.0, jnp.float32)
    zeros = jnp.zeros((16,), jnp.float32)

    @pl.loop(0, NCHUNK)
    def _chunk(c):
        pltpu.sync_copy(conn_hbm.at[pl.ds(tile_base + c * EPC, EPC)], idx_v)

        @pl.loop(0, STREAMS, unroll=8)
        def _fire(r):
            pltpu.async_copy(table_hbm.at[idx_v.at[pl.ds(r * 128, 128)]],
                             rows_v.at[pl.ds(r * 128, 128)], sem_rows)

        pltpu.make_async_copy(table_hbm.at[pl.ds(0, EPC)], rows_v,
                              sem_rows).wait()

        @pl.loop(0, GPC)
        def _group(g):
            gbase = jbase + g * (L * K * 2)
            sxx = zeros; sxy = zeros; sxz = zeros
            syy = zeros; syz = zeros; szz = zeros
            sdx = zeros; sdy = zeros; sdz = zeros
            for k in range(K):
                p0 = gbase + (2 * k)
                p1 = gbase + (2 * k + 1)
                x1 = plsc.load_gather(rows_v, [p0, c0])
                y1 = plsc.load_gather(rows_v, [p0, c1])
                z1 = plsc.load_gather(rows_v, [p0, c2])
                u1 = plsc.load_gather(rows_v, [p0, c3])
                x2 = plsc.load_gather(rows_v, [p1, c0])
                y2 = plsc.load_gather(rows_v, [p1, c1])
                z2 = plsc.load_gather(rows_v, [p1, c2])
                u2 = plsc.load_gather(rows_v, [p1, c3])
                dx = x1 - x2
                dy = y1 - y2
                dz = z1 - z2
                du = u1 - u2
                r2 = dx * dx + dy * dy + dz * dz
                w2 = jnp.where(r2 == 0.0, ones, ones / r2)
                tx = w2 * dx
                ty = w2 * dy
                tz = w2 * dz
                sxx += tx * dx; sxy += tx * dy; sxz += tx * dz
                sdx += tx * du
                syy += ty * dy; syz += ty * dz
                sdy += ty * du
                szz += tz * dz
                sdz += tz * du
            cof1 = syy * szz - syz * syz
            cof2 = sxy * szz - syz * sxz
            cof3 = sxy * syz - syy * sxz
            rdet = ones / (sxx * cof1 - sxy * cof2 + sxz * cof3)
            m1 = sdy * szz - syz * sdz
            m2 = sxy * sdz - sdy * sxz
            m3 = syy * sdz - sdy * syz
            nl = c * CB + g * L
            outx_v[pl.ds(nl, L)] = (sdx * cof1 - sxy * m1 + sxz * m3) * rdet
            outy_v[pl.ds(nl, L)] = (sxx * m1 - sdx * cof2 + sxz * m2) * rdet
            outz_v[pl.ds(nl, L)] = (sxx * m3 - sxy * m2 + sdx * cof3) * rdet

    obase = wid * PER_TILE
    pltpu.sync_copy(outx_v, outx_hbm.at[pl.ds(obase, PER_TILE)])
    pltpu.sync_copy(outy_v, outy_hbm.at[pl.ds(obase, PER_TILE)])
    pltpu.sync_copy(outz_v, outz_hbm.at[pl.ds(obase, PER_TILE)])


@jax.jit
def _run(conn_flat, table):
    mesh = plsc.VectorSubcoreMesh(core_axis_name="c", subcore_axis_name="s",
                                  num_cores=NC, num_subcores=NS)
    out = jax.ShapeDtypeStruct((NPAD,), jnp.float32)
    kfn = pl.kernel(
        _body,
        out_type=(out, out, out),
        mesh=mesh,
        scratch_types=[
            pltpu.VMEM((EPC,), jnp.int32),
            pltpu.VMEM((EPC, 4), jnp.float32),
            pltpu.VMEM((PER_TILE,), jnp.float32),
            pltpu.VMEM((PER_TILE,), jnp.float32),
            pltpu.VMEM((PER_TILE,), jnp.float32),
            pltpu.SemaphoreType.DMA,
        ],
    )
    return kfn(conn_flat, table)


def kernel(coords, connectivity_tensor, y):
    conn_flat = connectivity_tensor.reshape(-1)
    conn_flat = jnp.pad(conn_flat, (0, (NPAD - N) * K * 2))
    table = jnp.concatenate([coords, y], axis=1)
    dudx, dudy, dudz = _run(conn_flat, table)
    return (dudx[:N, None], dudy[:N, None], dudz[:N, None])


# speedup vs baseline: 19.5576x; 19.5576x over previous
import functools

import jax
import jax.numpy as jnp
from jax import lax
from jax.experimental import pallas as pl
from jax.experimental.pallas import tpu as pltpu
from jax.experimental.pallas import tpu_sc as plsc

N = 100000
K = 32
NC, NS, L = 2, 16, 16
NW = NC * NS
GROUPS = -(-N // (NW * L))
PER_TILE = GROUPS * L
NPAD = NW * PER_TILE
GPC = 7
CB = GPC * L
NCHUNK = GROUPS // GPC
EPC = CB * K * 2
STREAMS = EPC // 128


def _body(conn_hbm, table_hbm, outx_hbm, outy_hbm, outz_hbm,
          idx_v, rows_v, outx_v, outy_v, outz_v, sem_rows):
    wid = lax.axis_index("s") * NC + lax.axis_index("c")
    tile_base = wid * PER_TILE * K * 2

    jbase = lax.iota(jnp.int32, 16) * (K * 2)
    c0 = jnp.zeros((16,), jnp.int32)
    c1 = jnp.full((16,), 1, jnp.int32)
    c2 = jnp.full((16,), 2, jnp.int32)
    c3 = jnp.full((16,), 3, jnp.int32)
    ones = jnp.full((16,), 1.0, jnp.float32)
    zeros = jnp.zeros((16,), jnp.float32)

    @pl.loop(0, NCHUNK)
    def _chunk(c):
        pltpu.sync_copy(conn_hbm.at[pl.ds(tile_base + c * EPC, EPC)], idx_v)

        @pl.loop(0, STREAMS, unroll=8)
        def _fire(r):
            pltpu.async_copy(table_hbm.at[idx_v.at[pl.ds(r * 128, 128)]],
                             rows_v.at[pl.ds(r * 128, 128)], sem_rows)

        @pl.loop(0, STREAMS, unroll=8)
        def _drain(r):
            pltpu.make_async_copy(table_hbm.at[idx_v.at[pl.ds(r * 128, 128)]],
                                  rows_v.at[pl.ds(r * 128, 128)],
                                  sem_rows).wait()

        @pl.loop(0, GPC)
        def _group(g):
            gbase = jbase + g * (L * K * 2)
            sxx = zeros; sxy = zeros; sxz = zeros
            syy = zeros; syz = zeros; szz = zeros
            sdx = zeros; sdy = zeros; sdz = zeros
            for k in range(K):
                p0 = gbase + (2 * k)
                p1 = gbase + (2 * k + 1)
                x1 = plsc.load_gather(rows_v, [p0, c0])
                y1 = plsc.load_gather(rows_v, [p0, c1])
                z1 = plsc.load_gather(rows_v, [p0, c2])
                u1 = plsc.load_gather(rows_v, [p0, c3])
                x2 = plsc.load_gather(rows_v, [p1, c0])
                y2 = plsc.load_gather(rows_v, [p1, c1])
                z2 = plsc.load_gather(rows_v, [p1, c2])
                u2 = plsc.load_gather(rows_v, [p1, c3])
                dx = x1 - x2
                dy = y1 - y2
                dz = z1 - z2
                du = u1 - u2
                r2 = dx * dx + dy * dy + dz * dz
                w2 = jnp.where(r2 == 0.0, ones, ones / r2)
                tx = w2 * dx
                ty = w2 * dy
                tz = w2 * dz
                sxx += tx * dx; sxy += tx * dy; sxz += tx * dz
                sdx += tx * du
                syy += ty * dy; syz += ty * dz
                sdy += ty * du
                szz += tz * dz
                sdz += tz * du
            cof1 = syy * szz - syz * syz
            cof2 = sxy * szz - syz * sxz
            cof3 = sxy * syz - syy * sxz
            rdet = ones / (sxx * cof1 - sxy * cof2 + sxz * cof3)
            m1 = sdy * szz - syz * sdz
            m2 = sxy * sdz - sdy * sxz
            m3 = syy * sdz - sdy * syz
            nl = c * CB + g * L
            outx_v[pl.ds(nl, L)] = (sdx * cof1 - sxy * m1 + sxz * m3) * rdet
            outy_v[pl.ds(nl, L)] = (sxx * m1 - sdx * cof2 + sxz * m2) * rdet
            outz_v[pl.ds(nl, L)] = (sxx * m3 - sxy * m2 + sdx * cof3) * rdet

    obase = wid * PER_TILE
    pltpu.sync_copy(outx_v, outx_hbm.at[pl.ds(obase, PER_TILE)])
    pltpu.sync_copy(outy_v, outy_hbm.at[pl.ds(obase, PER_TILE)])
    pltpu.sync_copy(outz_v, outz_hbm.at[pl.ds(obase, PER_TILE)])


@jax.jit
def _run(conn_flat, table):
    mesh = plsc.VectorSubcoreMesh(core_axis_name="c", subcore_axis_name="s",
                                  num_cores=NC, num_subcores=NS)
    out = jax.ShapeDtypeStruct((NPAD,), jnp.float32)
    kfn = pl.kernel(
        _body,
        out_type=(out, out, out),
        mesh=mesh,
        compiler_params=pltpu.CompilerParams(needs_layout_passes=False,
                                             use_tc_tiling_on_sc=False),
        scratch_types=[
            pltpu.VMEM((EPC,), jnp.int32),
            pltpu.VMEM((EPC, 4), jnp.float32),
            pltpu.VMEM((PER_TILE,), jnp.float32),
            pltpu.VMEM((PER_TILE,), jnp.float32),
            pltpu.VMEM((PER_TILE,), jnp.float32),
            pltpu.SemaphoreType.DMA,
        ],
    )
    return kfn(conn_flat, table)


def kernel(coords, connectivity_tensor, y):
    conn_flat = connectivity_tensor.reshape(-1)
    conn_flat = jnp.pad(conn_flat, (0, (NPAD - N) * K * 2))
    table = jnp.concatenate([coords, y], axis=1)
    dudx, dudy, dudz = _run(conn_flat, table)
    return (dudx[:N, None], dudy[:N, None], dudz[:N, None])
